# baseline (speedup 1.0000x reference)
import functools

import jax
import jax.numpy as jnp
from jax import lax
from jax.experimental import pallas as pl
from jax.experimental.pallas import tpu as pltpu
from jax.experimental.pallas import tpu_sc as plsc

N = 100000
E = 1600000
D = 32
HD = 8
NC, NS, L = 2, 16, 16

BLK = 128
TB = 800
CHUNK = 200
NCH = TB // CHUNK
NBUF = 4
NW = CHUNK // NBUF
EB = NS * TB
E_PAD = EB * BLK

TB_D = 400
CHUNK_D = 16
NCH_D = TB_D // CHUNK_D

ACC_R = 102400
ZR = 400
NZC = 16
FL = ACC_R // NS

_mesh = plsc.VectorSubcoreMesh(
    core_axis_name="c", subcore_axis_name="s", num_cores=NC, num_subcores=NS)


def _zero_acc(acc, zbuf, tid):
  zbase = tid * (NZC * ZR)
  def body(k, _):
    pltpu.sync_copy(zbuf, acc.at[pl.ds(zbase + k * ZR, ZR)])
    return 0
  lax.fori_loop(0, NZC, body, 0)


def _flush(acc, o_hbm, tid):
  fbase = tid * FL
  pltpu.sync_copy(acc.at[pl.ds(fbase, FL)], o_hbm.at[pl.ds(fbase, FL)])


def _make_spmm(n_rounds):
  R = n_rounds
  outs = [jax.ShapeDtypeStruct((ACC_R, HD), jnp.float32) for _ in range(2 * R)]
  scratch = [
      pltpu.VMEM((CHUNK, BLK), jnp.int32),
      pltpu.VMEM((CHUNK, BLK), jnp.int32),
  ] + [pltpu.VMEM((BLK, HD), jnp.float32) for _ in range(NBUF)] + [
      pltpu.VMEM((ZR, HD), jnp.float32),
      pltpu.VMEM_SHARED((ACC_R, HD), jnp.float32),
  ] + [pltpu.SemaphoreType.DMA for _ in range(NBUF)]

  @functools.partial(pl.kernel, out_type=outs, mesh=_mesh,
                     scratch_types=scratch,
                     compiler_params=pltpu.CompilerParams(
                         use_tc_tiling_on_sc=False))
  def k(z_hbm, t_hbm, h_hbm, *rest):
    xs = rest[:2 * R]
    os = rest[2 * R:4 * R]
    sc = rest[4 * R:]
    tidx, hidx = sc[0], sc[1]
    rbufs = sc[2:2 + NBUF]
    zbuf, acc = sc[2 + NBUF], sc[3 + NBUF]
    sems = sc[4 + NBUF:4 + 2 * NBUF]
    cid = lax.axis_index("c")
    tid = lax.axis_index("s")
    pltpu.sync_copy(z_hbm, zbuf)

    def run_round(x_hbm, o_hbm):
      _zero_acc(acc, zbuf, tid)
      plsc.subcore_barrier()
      blk0 = tid * TB
      def chunk_body(ci, _):
        cb = blk0 + ci * CHUNK
        pltpu.sync_copy(t_hbm.at[pl.ds(cb, CHUNK)], tidx)
        pltpu.sync_copy(h_hbm.at[pl.ds(cb, CHUNK)], hidx)
        for b in range(NBUF):
          pltpu.async_copy(x_hbm.at[tidx.at[b]], rbufs[b], sems[b])
        def win_body(w, _):
          for b in range(NBUF):
            j = w * NBUF + b
            pltpu.make_async_copy(
                x_hbm.at[pl.ds(0, BLK)], rbufs[b], sems[b]).wait()
            pltpu.sync_copy(rbufs[b], acc.at[hidx.at[j]], add=True)
            @pl.when(w < NW - 1)
            def _():
              pltpu.async_copy(x_hbm.at[tidx.at[j + NBUF]], rbufs[b], sems[b])
          return 0
        lax.fori_loop(0, NW, win_body, 0)
        return 0
      lax.fori_loop(0, NCH, chunk_body, 0)
      plsc.subcore_barrier()
      _flush(acc, o_hbm, tid)
      plsc.subcore_barrier()

    @pl.when(cid == 0)
    def _():
      for r in range(R):
        run_round(xs[2 * r], os[2 * r])

    @pl.when(cid == 1)
    def _():
      for r in range(R):
        run_round(xs[2 * r + 1], os[2 * r + 1])

  return k


def _make_deg():
  outs = [jax.ShapeDtypeStruct((ACC_R, HD), jnp.float32) for _ in range(2)]
  scratch = [
      pltpu.VMEM((CHUNK_D, BLK), jnp.int32),
      pltpu.VMEM((BLK, HD), jnp.float32),
      pltpu.VMEM((ZR, HD), jnp.float32),
      pltpu.VMEM_SHARED((ACC_R, HD), jnp.float32),
  ]

  @functools.partial(pl.kernel, out_type=outs, mesh=_mesh,
                     scratch_types=scratch,
                     compiler_params=pltpu.CompilerParams(
                         use_tc_tiling_on_sc=False))
  def k(z_hbm, ones_hbm, h_hbm, o0_hbm, o1_hbm, hidx, ones, zbuf, acc):
    cid = lax.axis_index("c")
    tid = lax.axis_index("s")
    pltpu.sync_copy(z_hbm, zbuf)
    pltpu.sync_copy(ones_hbm, ones)

    _zero_acc(acc, zbuf, tid)
    plsc.subcore_barrier()
    blk0 = cid * (NS * TB_D) + tid * TB_D
    def chunk_body(ci, _):
      pltpu.sync_copy(h_hbm.at[pl.ds(blk0 + ci * CHUNK_D, CHUNK_D)], hidx)
      def blk_body(j, _):
        pltpu.sync_copy(ones, acc.at[hidx.at[j]], add=True)
        return 0
      lax.fori_loop(0, CHUNK_D, blk_body, 0)
      return 0
    lax.fori_loop(0, NCH_D, chunk_body, 0)
    plsc.subcore_barrier()

    @pl.when(cid == 0)
    def _():
      _flush(acc, o0_hbm, tid)

    @pl.when(cid == 1)
    def _():
      _flush(acc, o1_hbm, tid)

  return k


_spmm2 = _make_spmm(2)
_spmm6 = _make_spmm(6)
_deg = _make_deg()


NP = ACC_R
BR = 2048
BRV = BR * HD // 128
GRID = NP // BR
VR = ACC_R * HD // 128

_q = pl.BlockSpec((BRV, 128), lambda i: (i, 0))
_f = pl.BlockSpec((BR, D), lambda i: (i, 0))
_qs = jax.ShapeDtypeStruct((VR, 128), jnp.float32)


def _seg_norm_rep(sq, seg):
  s = jnp.dot(sq, seg[...], preferred_element_type=jnp.float32,
              precision=lax.Precision.HIGHEST)
  return jnp.maximum(jnp.sqrt(s), 1e-12)


def _tc1_body(d0, d1, *refs):
  embq = refs[0:4]
  dinv_o = refs[4]
  e_o = refs[5:9]
  deg = d0[...] + d1[...]
  dv = jnp.where(deg > 0, lax.rsqrt(deg), 0.0)
  dinv_o[...] = dv
  for q in range(4):
    e_o[q][...] = embq[q][...] * dv


def _tc1(deg0v, deg1v, embqv):
  return pl.pallas_call(
      _tc1_body,
      grid=(GRID,),
      in_specs=[_q] * 6,
      out_specs=[_q] * 5,
      out_shape=[_qs] * 5,
  )(deg0v, deg1v, *embqv)


def _tc2_body(*refs):
  uq = refs[0:4]
  s = refs[4][...]
  n1q = [r[...] for r in refs[5:9]]
  n2q = [r[...] for r in refs[9:13]]
  seg = refs[13]
  outs = refs[14:26]
  s2 = s * s
  f1 = 0.2 / _seg_norm_rep(sum(n * n for n in n1q), seg)
  f2 = 0.2 / _seg_norm_rep(sum(n * n for n in n2q), seg)
  for q in range(4):
    uv = uq[q][...]
    ys = s2 * uv
    sg = jnp.sign(uv) * s
    outs[q][...] = ys
    outs[4 + q][...] = ys + n1q[q] * f1 * sg
    outs[8 + q][...] = ys + n2q[q] * f2 * sg


def _tc2(u1q, dinvv, n1q, n2q, seg):
  return pl.pallas_call(
      _tc2_body,
      grid=(GRID,),
      in_specs=[_q] * 13 + [pl.BlockSpec((128, 128), lambda i: (0, 0))],
      out_specs=[_q] * 12,
      out_shape=[_qs] * 12,
  )(*u1q, dinvv, *n1q, *n2q, seg)


def _tc3_body(*refs):
  uq = refs[0:4]
  vq = refs[4:8]
  vaq = refs[8:12]
  vbq = refs[12:16]
  s = refs[16][...]
  nq = [[r[...] for r in refs[17 + 4 * i:21 + 4 * i]] for i in range(4)]
  seg = refs[33]
  embq = refs[34:38]
  outs = refs[38:50]
  fs = [0.2 / _seg_norm_rep(sum(n * n for n in nqi), seg) for nqi in nq]
  for q in range(4):
    y = s * uq[q][...]
    p1 = y + nq[0][q] * fs[0] * jnp.sign(y)
    p2 = y + nq[1][q] * fs[1] * jnp.sign(y)
    z = s * vq[q][...]
    z1 = s * vaq[q][...]
    z2 = s * vbq[q][...]
    e = embq[q][...]
    outs[q][...] = e + y + z
    outs[4 + q][...] = e + p1 + (z1 + nq[2][q] * fs[2] * jnp.sign(z1))
    outs[8 + q][...] = e + p2 + (z2 + nq[3][q] * fs[3] * jnp.sign(z2))


def _tc3(embqv, u1q, vq, vaq, vbq, dinvv, nqs, seg):
  return pl.pallas_call(
      _tc3_body,
      grid=(GRID,),
      in_specs=[_q] * 33 + [pl.BlockSpec((128, 128), lambda i: (0, 0))]
      + [_q] * 4,
      out_specs=[_q] * 12,
      out_shape=[_qs] * 12,
  )(*u1q, *vq, *vaq, *vbq, dinvv, *nqs[0], *nqs[1], *nqs[2], *nqs[3],
    seg, *embqv)


def kernel(user_emb, item_emb, all_h_list, all_t_list):
  emb0 = jnp.concatenate([user_emb, item_emb], axis=0)
  embp = jnp.pad(emb0, ((0, NP - N), (0, 0)))
  h = all_h_list.astype(jnp.int32)
  t = all_t_list.astype(jnp.int32)
  pad = E_PAD - E
  h2d = jnp.concatenate([h, jnp.full((pad,), N, jnp.int32)]).reshape(EB, BLK)
  t2d = jnp.concatenate([t, jnp.zeros((pad,), jnp.int32)]).reshape(EB, BLK)

  key = jax.random.key(42)
  key, ka, kb = jax.random.split(key, 3)
  n1 = jax.random.uniform(ka, (N, D), dtype=jnp.float32)
  n2 = jax.random.uniform(kb, (N, D), dtype=jnp.float32)
  key, kc, kd = jax.random.split(key, 3)
  n3 = jax.random.uniform(kc, (N, D), dtype=jnp.float32)
  n4 = jax.random.uniform(kd, (N, D), dtype=jnp.float32)
  padn = lambda a: jnp.pad(a, ((0, NP - N), (0, 0)))
  n1, n2, n3, n4 = padn(n1), padn(n2), padn(n3), padn(n4)

  zconst = jnp.zeros((ZR, HD), jnp.float32)
  oconst = jnp.ones((BLK, HD), jnp.float32)

  pv = lambda a: a.reshape(VR, 128)
  un = lambda a: a.reshape(ACC_R, HD)
  qsplit = lambda a: list(
      a.reshape(NP, 4, HD).transpose(1, 0, 2).reshape(4, VR, 128))
  seg = jnp.kron(jnp.eye(16, dtype=jnp.float32),
                 jnp.ones((8, 8), jnp.float32))

  embq = qsplit(embp)
  n1q, n2q, n3q, n4q = qsplit(n1), qsplit(n2), qsplit(n3), qsplit(n4)

  deg0, deg1 = _deg(zconst, oconst, h2d)
  dinvv, e0, e1, e2, e3 = _tc1(pv(deg0), pv(deg1), embq)
  u0, u2, u1, u3 = _spmm2(zconst, t2d, h2d, un(e0), un(e2), un(e1), un(e3))
  o = _tc2([pv(u0), pv(u1), pv(u2), pv(u3)], dinvv, n1q, n2q, seg)
  ys, p1s, p2s = o[0:4], o[4:8], o[8:12]
  (v0, v2, v1, v3,
   va0, va2, va1, va3,
   vb0, vb2, vb1, vb3) = _spmm6(
      zconst, t2d, h2d,
      un(ys[0]), un(ys[2]), un(ys[1]), un(ys[3]),
      un(p1s[0]), un(p1s[2]), un(p1s[1]), un(p1s[3]),
      un(p2s[0]), un(p2s[2]), un(p2s[1]), un(p2s[3]))
  ob = _tc3(embq, [pv(u0), pv(u1), pv(u2), pv(u3)],
            [pv(v0), pv(v1), pv(v2), pv(v3)],
            [pv(va0), pv(va1), pv(va2), pv(va3)],
            [pv(vb0), pv(vb1), pv(vb2), pv(vb3)],
            dinvv, [n1q, n2q, n3q, n4q], seg)
  qmerge = lambda qs: jnp.stack(qs).reshape(4, NP, HD).transpose(
      1, 0, 2).reshape(NP, D)[:N]
  return jnp.stack([qmerge(ob[0:4]), qmerge(ob[4:8]), qmerge(ob[8:12])])


# speedup vs baseline: 1.0621x; 1.0621x over previous
import functools

import jax
import jax.numpy as jnp
from jax import lax
from jax.experimental import pallas as pl
from jax.experimental.pallas import tpu as pltpu
from jax.experimental.pallas import tpu_sc as plsc

N = 100000
E = 1600000
D = 32
HD = 8
NC, NS, L = 2, 16, 16

BLK = 128
TB = 800
CHUNK = 200
NCH = TB // CHUNK
NBUF = 8
NW = CHUNK // NBUF
EB = NS * TB
E_PAD = EB * BLK

TB_D = 400
CHUNK_D = 16
NCH_D = TB_D // CHUNK_D

ACC_R = 102400
ZR = 400
NZC = 16
FL = ACC_R // NS

_mesh = plsc.VectorSubcoreMesh(
    core_axis_name="c", subcore_axis_name="s", num_cores=NC, num_subcores=NS)


def _zero_acc(acc, zbuf, tid):
  zbase = tid * (NZC * ZR)
  def body(k, _):
    pltpu.sync_copy(zbuf, acc.at[pl.ds(zbase + k * ZR, ZR)])
    return 0
  lax.fori_loop(0, NZC, body, 0)


def _flush(acc, o_hbm, tid):
  fbase = tid * FL
  pltpu.sync_copy(acc.at[pl.ds(fbase, FL)], o_hbm.at[pl.ds(fbase, FL)])


def _make_spmm(n_rounds):
  R = n_rounds
  outs = [jax.ShapeDtypeStruct((ACC_R, HD), jnp.float32) for _ in range(2 * R)]
  scratch = [
      pltpu.VMEM((CHUNK, BLK), jnp.int32),
      pltpu.VMEM((CHUNK, BLK), jnp.int32),
  ] + [pltpu.VMEM((BLK, HD), jnp.float32) for _ in range(NBUF)] + [
      pltpu.VMEM((ZR, HD), jnp.float32),
      pltpu.VMEM_SHARED((ACC_R, HD), jnp.float32),
  ] + [pltpu.SemaphoreType.DMA for _ in range(NBUF)]

  @functools.partial(pl.kernel, out_type=outs, mesh=_mesh,
                     scratch_types=scratch,
                     compiler_params=pltpu.CompilerParams(
                         use_tc_tiling_on_sc=False))
  def k(z_hbm, t_hbm, h_hbm, *rest):
    xs = rest[:2 * R]
    os = rest[2 * R:4 * R]
    sc = rest[4 * R:]
    tidx, hidx = sc[0], sc[1]
    rbufs = sc[2:2 + NBUF]
    zbuf, acc = sc[2 + NBUF], sc[3 + NBUF]
    sems = sc[4 + NBUF:4 + 2 * NBUF]
    cid = lax.axis_index("c")
    tid = lax.axis_index("s")
    pltpu.sync_copy(z_hbm, zbuf)

    def run_round(x_hbm, o_hbm):
      _zero_acc(acc, zbuf, tid)
      plsc.subcore_barrier()
      blk0 = tid * TB
      def chunk_body(ci, _):
        cb = blk0 + ci * CHUNK
        pltpu.sync_copy(t_hbm.at[pl.ds(cb, CHUNK)], tidx)
        pltpu.sync_copy(h_hbm.at[pl.ds(cb, CHUNK)], hidx)
        for b in range(NBUF):
          pltpu.async_copy(x_hbm.at[tidx.at[b]], rbufs[b], sems[b])
        def win_body(w, _):
          for b in range(NBUF):
            j = w * NBUF + b
            pltpu.make_async_copy(
                x_hbm.at[pl.ds(0, BLK)], rbufs[b], sems[b]).wait()
            pltpu.sync_copy(rbufs[b], acc.at[hidx.at[j]], add=True)
            @pl.when(w < NW - 1)
            def _():
              pltpu.async_copy(x_hbm.at[tidx.at[j + NBUF]], rbufs[b], sems[b])
          return 0
        lax.fori_loop(0, NW, win_body, 0)
        return 0
      lax.fori_loop(0, NCH, chunk_body, 0)
      plsc.subcore_barrier()
      _flush(acc, o_hbm, tid)
      plsc.subcore_barrier()

    @pl.when(cid == 0)
    def _():
      for r in range(R):
        run_round(xs[2 * r], os[2 * r])

    @pl.when(cid == 1)
    def _():
      for r in range(R):
        run_round(xs[2 * r + 1], os[2 * r + 1])

  return k


def _make_deg():
  outs = [jax.ShapeDtypeStruct((ACC_R, HD), jnp.float32) for _ in range(2)]
  scratch = [
      pltpu.VMEM((CHUNK_D, BLK), jnp.int32),
      pltpu.VMEM((BLK, HD), jnp.float32),
      pltpu.VMEM((ZR, HD), jnp.float32),
      pltpu.VMEM_SHARED((ACC_R, HD), jnp.float32),
  ]

  @functools.partial(pl.kernel, out_type=outs, mesh=_mesh,
                     scratch_types=scratch,
                     compiler_params=pltpu.CompilerParams(
                         use_tc_tiling_on_sc=False))
  def k(z_hbm, ones_hbm, h_hbm, o0_hbm, o1_hbm, hidx, ones, zbuf, acc):
    cid = lax.axis_index("c")
    tid = lax.axis_index("s")
    pltpu.sync_copy(z_hbm, zbuf)
    pltpu.sync_copy(ones_hbm, ones)

    _zero_acc(acc, zbuf, tid)
    plsc.subcore_barrier()
    blk0 = cid * (NS * TB_D) + tid * TB_D
    def chunk_body(ci, _):
      pltpu.sync_copy(h_hbm.at[pl.ds(blk0 + ci * CHUNK_D, CHUNK_D)], hidx)
      def blk_body(j, _):
        pltpu.sync_copy(ones, acc.at[hidx.at[j]], add=True)
        return 0
      lax.fori_loop(0, CHUNK_D, blk_body, 0)
      return 0
    lax.fori_loop(0, NCH_D, chunk_body, 0)
    plsc.subcore_barrier()

    @pl.when(cid == 0)
    def _():
      _flush(acc, o0_hbm, tid)

    @pl.when(cid == 1)
    def _():
      _flush(acc, o1_hbm, tid)

  return k


_spmm2 = _make_spmm(2)
_spmm6 = _make_spmm(6)
_deg = _make_deg()


NP = ACC_R
BR = 2048
BRV = BR * HD // 128
GRID = NP // BR
VR = ACC_R * HD // 128

_q = pl.BlockSpec((BRV, 128), lambda i: (i, 0))
_f = pl.BlockSpec((BR, D), lambda i: (i, 0))
_qs = jax.ShapeDtypeStruct((VR, 128), jnp.float32)


def _seg_norm_rep(sq, seg):
  s = jnp.dot(sq, seg[...], preferred_element_type=jnp.float32,
              precision=lax.Precision.HIGHEST)
  return jnp.maximum(jnp.sqrt(s), 1e-12)


def _tc1_body(d0, d1, *refs):
  embq = refs[0:4]
  dinv_o = refs[4]
  e_o = refs[5:9]
  deg = d0[...] + d1[...]
  dv = jnp.where(deg > 0, lax.rsqrt(deg), 0.0)
  dinv_o[...] = dv
  for q in range(4):
    e_o[q][...] = embq[q][...] * dv


def _tc1(deg0v, deg1v, embqv):
  return pl.pallas_call(
      _tc1_body,
      grid=(GRID,),
      in_specs=[_q] * 6,
      out_specs=[_q] * 5,
      out_shape=[_qs] * 5,
  )(deg0v, deg1v, *embqv)


def _tc2_body(*refs):
  uq = refs[0:4]
  s = refs[4][...]
  n1q = [r[...] for r in refs[5:9]]
  n2q = [r[...] for r in refs[9:13]]
  seg = refs[13]
  outs = refs[14:26]
  s2 = s * s
  f1 = 0.2 / _seg_norm_rep(sum(n * n for n in n1q), seg)
  f2 = 0.2 / _seg_norm_rep(sum(n * n for n in n2q), seg)
  for q in range(4):
    uv = uq[q][...]
    ys = s2 * uv
    sg = jnp.sign(uv) * s
    outs[q][...] = ys
    outs[4 + q][...] = ys + n1q[q] * f1 * sg
    outs[8 + q][...] = ys + n2q[q] * f2 * sg


def _tc2(u1q, dinvv, n1q, n2q, seg):
  return pl.pallas_call(
      _tc2_body,
      grid=(GRID,),
      in_specs=[_q] * 13 + [pl.BlockSpec((128, 128), lambda i: (0, 0))],
      out_specs=[_q] * 12,
      out_shape=[_qs] * 12,
  )(*u1q, dinvv, *n1q, *n2q, seg)


def _tc3_body(*refs):
  uq = refs[0:4]
  vq = refs[4:8]
  vaq = refs[8:12]
  vbq = refs[12:16]
  s = refs[16][...]
  nq = [[r[...] for r in refs[17 + 4 * i:21 + 4 * i]] for i in range(4)]
  seg = refs[33]
  embq = refs[34:38]
  outs = refs[38:50]
  fs = [0.2 / _seg_norm_rep(sum(n * n for n in nqi), seg) for nqi in nq]
  for q in range(4):
    y = s * uq[q][...]
    p1 = y + nq[0][q] * fs[0] * jnp.sign(y)
    p2 = y + nq[1][q] * fs[1] * jnp.sign(y)
    z = s * vq[q][...]
    z1 = s * vaq[q][...]
    z2 = s * vbq[q][...]
    e = embq[q][...]
    outs[q][...] = e + y + z
    outs[4 + q][...] = e + p1 + (z1 + nq[2][q] * fs[2] * jnp.sign(z1))
    outs[8 + q][...] = e + p2 + (z2 + nq[3][q] * fs[3] * jnp.sign(z2))


def _tc3(embqv, u1q, vq, vaq, vbq, dinvv, nqs, seg):
  return pl.pallas_call(
      _tc3_body,
      grid=(GRID,),
      in_specs=[_q] * 33 + [pl.BlockSpec((128, 128), lambda i: (0, 0))]
      + [_q] * 4,
      out_specs=[_q] * 12,
      out_shape=[_qs] * 12,
  )(*u1q, *vq, *vaq, *vbq, dinvv, *nqs[0], *nqs[1], *nqs[2], *nqs[3],
    seg, *embqv)


def kernel(user_emb, item_emb, all_h_list, all_t_list):
  emb0 = jnp.concatenate([user_emb, item_emb], axis=0)
  embp = jnp.pad(emb0, ((0, NP - N), (0, 0)))
  h = all_h_list.astype(jnp.int32)
  t = all_t_list.astype(jnp.int32)
  pad = E_PAD - E
  h2d = jnp.concatenate([h, jnp.full((pad,), N, jnp.int32)]).reshape(EB, BLK)
  t2d = jnp.concatenate([t, jnp.zeros((pad,), jnp.int32)]).reshape(EB, BLK)

  key = jax.random.key(42)
  key, ka, kb = jax.random.split(key, 3)
  n1 = jax.random.uniform(ka, (N, D), dtype=jnp.float32)
  n2 = jax.random.uniform(kb, (N, D), dtype=jnp.float32)
  key, kc, kd = jax.random.split(key, 3)
  n3 = jax.random.uniform(kc, (N, D), dtype=jnp.float32)
  n4 = jax.random.uniform(kd, (N, D), dtype=jnp.float32)
  padn = lambda a: jnp.pad(a, ((0, NP - N), (0, 0)))
  n1, n2, n3, n4 = padn(n1), padn(n2), padn(n3), padn(n4)

  zconst = jnp.zeros((ZR, HD), jnp.float32)
  oconst = jnp.ones((BLK, HD), jnp.float32)

  pv = lambda a: a.reshape(VR, 128)
  un = lambda a: a.reshape(ACC_R, HD)
  qsplit = lambda a: list(
      a.reshape(NP, 4, HD).transpose(1, 0, 2).reshape(4, VR, 128))
  seg = jnp.kron(jnp.eye(16, dtype=jnp.float32),
                 jnp.ones((8, 8), jnp.float32))

  embq = qsplit(embp)
  n1q, n2q, n3q, n4q = qsplit(n1), qsplit(n2), qsplit(n3), qsplit(n4)

  deg0, deg1 = _deg(zconst, oconst, h2d)
  dinvv, e0, e1, e2, e3 = _tc1(pv(deg0), pv(deg1), embq)
  u0, u2, u1, u3 = _spmm2(zconst, t2d, h2d, un(e0), un(e2), un(e1), un(e3))
  o = _tc2([pv(u0), pv(u1), pv(u2), pv(u3)], dinvv, n1q, n2q, seg)
  ys, p1s, p2s = o[0:4], o[4:8], o[8:12]
  (v0, v2, v1, v3,
   va0, va2, va1, va3,
   vb0, vb2, vb1, vb3) = _spmm6(
      zconst, t2d, h2d,
      un(ys[0]), un(ys[2]), un(ys[1]), un(ys[3]),
      un(p1s[0]), un(p1s[2]), un(p1s[1]), un(p1s[3]),
      un(p2s[0]), un(p2s[2]), un(p2s[1]), un(p2s[3]))
  ob = _tc3(embq, [pv(u0), pv(u1), pv(u2), pv(u3)],
            [pv(v0), pv(v1), pv(v2), pv(v3)],
            [pv(va0), pv(va1), pv(va2), pv(va3)],
            [pv(vb0), pv(vb1), pv(vb2), pv(vb3)],
            dinvv, [n1q, n2q, n3q, n4q], seg)
  qmerge = lambda qs: jnp.stack(qs).reshape(4, NP, HD).transpose(
      1, 0, 2).reshape(NP, D)[:N]
  return jnp.stack([qmerge(ob[0:4]), qmerge(ob[4:8]), qmerge(ob[8:12])])
